# baseline (speedup 1.0000x reference)
import functools

import jax
import jax.numpy as jnp
from jax import lax
from jax.experimental import pallas as pl
from jax.experimental.pallas import tpu as pltpu
from jax.experimental.pallas import tpu_sc as plsc

T = 2048
H = 1024
DFF = 2048
E = 8
TILE = 128
P = T * 2 + E * TILE
NTILES = P // TILE
NC, NS, LANES = 2, 16, 16
NW = NC * NS
TPW = T // NW
HALF = TPW // 2

_NEG_INF = float("-inf")


def _routing_body(hid_ref, wgate_ref, pe_ref, po_ref, se_ref, so_ref,
                  g_ref, nv_ref, b_ref, nx_ref):
    hid = hid_ref[...]
    logits = lax.dot_general(hid, wgate_ref[...], (((1,), (1,)), ((), ())),
                             preferred_element_type=jnp.float32)
    iota_e = lax.broadcasted_iota(jnp.int32, (T, E), 1)
    m1 = jnp.max(logits, axis=1, keepdims=True)
    am1 = jnp.min(jnp.where(logits == m1, iota_e, E), axis=1, keepdims=True)
    masked = jnp.where(iota_e == am1, _NEG_INF, logits)
    m2 = jnp.max(masked, axis=1, keepdims=True)
    am2 = jnp.min(jnp.where(masked == m2, iota_e, E), axis=1, keepdims=True)
    s1 = 1.0 / (1.0 + jnp.exp(m2 - m1))
    s2 = 1.0 - s1

    one_e = (iota_e == am1).astype(jnp.float32)
    one_o = (iota_e == am2).astype(jnp.float32)
    segment = one_e + one_o
    incl = segment
    shift = 1
    while shift < T:
        top = jnp.zeros((shift, E), jnp.float32)
        incl = incl + jnp.concatenate([top, incl[: T - shift]], axis=0)
        shift *= 2
    run_excl = incl - segment

    counts = jnp.sum(segment, axis=0, keepdims=True)
    pc = ((counts.astype(jnp.int32) + (TILE - 1)) // TILE) * TILE
    re8 = lax.broadcasted_iota(jnp.int32, (E, E), 0)
    ce8 = lax.broadcasted_iota(jnp.int32, (E, E), 1)
    upper = (re8 < ce8).astype(jnp.float32)
    pstart = lax.dot_general(pc.astype(jnp.float32), upper,
                             (((1,), (0,)), ((), ())),
                             preferred_element_type=jnp.float32)
    total = jnp.sum(pc)

    slot = pstart + run_excl
    pe_ref[...] = jnp.sum(one_e * slot, axis=1, keepdims=True).astype(jnp.int32)
    po_ref[...] = jnp.sum(one_o * slot, axis=1, keepdims=True).astype(jnp.int32)
    se_ref[...] = jnp.broadcast_to(s1, (T, LANES))
    so_ref[...] = jnp.broadcast_to(s2, (T, LANES))

    tstart = lax.broadcasted_iota(jnp.int32, (NTILES, E), 0) * TILE
    teff = jnp.minimum(tstart, total - TILE)
    ipstart = pstart.astype(jnp.int32)
    g = jnp.sum((teff >= ipstart).astype(jnp.int32), axis=1, keepdims=True) - 1
    g_ref[...] = g
    nv_ref[...] = jnp.full((8, 1), total // TILE, jnp.int32)

    iota_ne = lax.broadcasted_iota(jnp.int32, (NTILES, E), 1)
    used = jnp.broadcast_to((pc > 0).astype(jnp.int32), (NTILES, E))
    b_ref[...] = jnp.sum(used * (iota_ne < g).astype(jnp.int32),
                         axis=1, keepdims=True)
    cand = jnp.where((used > 0) & (iota_ne > g), iota_ne, E + 99)
    nxt = jnp.min(cand, axis=1, keepdims=True)
    nx_ref[...] = jnp.where(nxt == E + 99, -1, nxt)


def _routing_call(hidden_states, w_gate, interpret=False):
    return pl.pallas_call(
        _routing_body,
        out_shape=(
            jax.ShapeDtypeStruct((T, 1), jnp.int32),
            jax.ShapeDtypeStruct((T, 1), jnp.int32),
            jax.ShapeDtypeStruct((T, LANES), jnp.float32),
            jax.ShapeDtypeStruct((T, LANES), jnp.float32),
            jax.ShapeDtypeStruct((NTILES, 1), jnp.int32),
            jax.ShapeDtypeStruct((8, 1), jnp.int32),
            jax.ShapeDtypeStruct((NTILES, 1), jnp.int32),
            jax.ShapeDtypeStruct((NTILES, 1), jnp.int32),
        ),
        interpret=interpret,
    )(hidden_states, w_gate)


def _ffn_body(g_sref, nv_sref, b_sref, nx_sref,
              x_ref, wg_hbm, wu_hbm, w2_hbm, out_ref,
              wgb, wub, w2b, sg0, su0, s20, sg1, su1, s21):
    t = pl.program_id(0)
    nv = nv_sref[0]
    e = g_sref[t]
    ordb = b_sref[t]
    is_valid = t < nv
    prevb = b_sref[jnp.maximum(t - 1, 0)]
    at_boundary = jnp.logical_and(is_valid,
                                  jnp.logical_or(t == 0, ordb != prevb))
    sems = ((sg0, su0, s20), (sg1, su1, s21))

    def _halves(eid, k):
        for hbm, buf, sem in ((wg_hbm, wgb, sems[k][0]),
                              (wu_hbm, wub, sems[k][1]),
                              (w2_hbm, w2b, sems[k][2])):
            m = hbm.shape[1] // 2
            yield pltpu.make_async_copy(hbm.at[eid, pl.ds(0, m)],
                                        buf.at[k, pl.ds(0, m)], sem)
            yield pltpu.make_async_copy(hbm.at[eid, pl.ds(m, m)],
                                        buf.at[k, pl.ds(m, m)], sem)

    def start_fetch(eid, k):
        for cp in _halves(eid, k):
            cp.start()

    @pl.when(jnp.logical_and(t == 0, is_valid))
    def _():
        start_fetch(e, 0)

    for k in (0, 1):
        @pl.when(jnp.logical_and(at_boundary, lax.rem(ordb, 2) == k))
        def _(k=k):
            for cp in _halves(e, k):
                cp.wait()
            ne = nx_sref[t]

            @pl.when(ne >= 0)
            def _():
                start_fetch(ne, 1 - k)

    for k in (0, 1):
        @pl.when(jnp.logical_and(is_valid, lax.rem(ordb, 2) == k))
        def _(k=k):
            x = x_ref[...]
            a = jnp.dot(x, wgb[k], preferred_element_type=jnp.float32)
            bb = jnp.dot(x, wub[k], preferred_element_type=jnp.float32)
            h = a * (1.0 / (1.0 + jnp.exp(-a))) * bb
            out_ref[...] = jnp.dot(h, w2b[k], preferred_element_type=jnp.float32)


def _ffn_call(g, nv, b, nx, x_sorted, wg, wu, w2, interpret=False):
    grid_spec = pltpu.PrefetchScalarGridSpec(
        num_scalar_prefetch=4,
        grid=(NTILES,),
        in_specs=[
            pl.BlockSpec((TILE, H),
                         lambda t, g, nv, b, nx: (jnp.minimum(t, nv[0] - 1), 0)),
            pl.BlockSpec(memory_space=pl.ANY),
            pl.BlockSpec(memory_space=pl.ANY),
            pl.BlockSpec(memory_space=pl.ANY),
        ],
        out_specs=pl.BlockSpec((TILE, H),
                               lambda t, g, nv, b, nx: (jnp.minimum(t, nv[0] - 1), 0)),
        scratch_shapes=[
            pltpu.VMEM((2, H, DFF), jnp.float32),
            pltpu.VMEM((2, H, DFF), jnp.float32),
            pltpu.VMEM((2, DFF, H), jnp.float32),
            pltpu.SemaphoreType.DMA,
            pltpu.SemaphoreType.DMA,
            pltpu.SemaphoreType.DMA,
            pltpu.SemaphoreType.DMA,
            pltpu.SemaphoreType.DMA,
            pltpu.SemaphoreType.DMA,
        ],
    )
    return pl.pallas_call(
        _ffn_body,
        grid_spec=grid_spec,
        out_shape=jax.ShapeDtypeStruct((P, H), jnp.float32),
        name="moe_ffn",
        compiler_params=pltpu.CompilerParams(
            dimension_semantics=("arbitrary",)),
        interpret=interpret,
    )(g, nv, b, nx, x_sorted, wg, wu, w2)


def _dispatch_body(hid_hbm, pe_hbm, po_hbm, xs_hbm, idx0, idx1, rows, sem):
    wid = lax.axis_index("s") * NC + lax.axis_index("c")
    base = wid * TPW
    pltpu.sync_copy(pe_hbm.at[pl.ds(base, TPW)], idx0)
    pltpu.sync_copy(po_hbm.at[pl.ds(base, TPW)], idx1)
    pltpu.sync_copy(hid_hbm.at[pl.ds(base, TPW)], rows)
    cp0 = pltpu.async_copy(rows, xs_hbm.at[idx0], sem)
    cp1 = pltpu.async_copy(rows, xs_hbm.at[idx1], sem)
    cp0.wait()
    cp1.wait()


@functools.cache
def _dispatch_kernel():
    return pl.kernel(
        _dispatch_body,
        mesh=plsc.VectorSubcoreMesh(core_axis_name="c", subcore_axis_name="s"),
        out_type=jax.ShapeDtypeStruct((P, H), jnp.float32),
        scratch_types=[
            pltpu.VMEM((TPW,), jnp.int32),
            pltpu.VMEM((TPW,), jnp.int32),
            pltpu.VMEM((TPW, H), jnp.float32),
            pltpu.SemaphoreType.DMA,
        ],
    )


def _combine_body(os_hbm, pe_hbm, po_hbm, se_hbm, so_hbm, out_hbm,
                  idx0a, idx1a, idx0b, idx1b, rows0, rows1, sev, sov, acc,
                  sem0, sem1):
    wid = lax.axis_index("s") * NC + lax.axis_index("c")
    ha = wid * TPW
    hb = ha + HALF
    pltpu.sync_copy(pe_hbm.at[pl.ds(ha, HALF)], idx0a)
    pltpu.sync_copy(po_hbm.at[pl.ds(ha, HALF)], idx1a)
    pltpu.sync_copy(pe_hbm.at[pl.ds(hb, HALF)], idx0b)
    pltpu.sync_copy(po_hbm.at[pl.ds(hb, HALF)], idx1b)
    a0 = pltpu.async_copy(os_hbm.at[idx0a], rows0, sem0)
    a1 = pltpu.async_copy(os_hbm.at[idx1a], rows1, sem1)
    pltpu.sync_copy(se_hbm.at[pl.ds(ha, 2 * HALF)], sev)
    pltpu.sync_copy(so_hbm.at[pl.ds(ha, 2 * HALF)], sov)

    def mul_into(dst, src, sc_ref, off, accumulate):
        def tok_body(tk, carry):
            s = sc_ref[off + tk]
            for cch in range(H // LANES):
                sl = pl.ds(cch * LANES, LANES)
                if accumulate:
                    dst[tk, sl] = dst[tk, sl] + src[tk, sl] * s
                else:
                    dst[tk, sl] = src[tk, sl] * s
            return carry
        lax.fori_loop(0, HALF, tok_body, 0)

    a0.wait()
    mul_into(acc, rows0, sev, 0, False)
    b0 = pltpu.async_copy(os_hbm.at[idx0b], rows0, sem0)
    a1.wait()
    mul_into(acc, rows1, sov, 0, True)
    b1 = pltpu.async_copy(os_hbm.at[idx1b], rows1, sem1)
    pltpu.sync_copy(acc, out_hbm.at[pl.ds(ha, HALF)])
    b0.wait()
    mul_into(acc, rows0, sev, HALF, False)
    b1.wait()
    mul_into(acc, rows1, sov, HALF, True)
    pltpu.sync_copy(acc, out_hbm.at[pl.ds(hb, HALF)])


@functools.cache
def _combine_kernel():
    return pl.kernel(
        _combine_body,
        mesh=plsc.VectorSubcoreMesh(core_axis_name="c", subcore_axis_name="s"),
        out_type=jax.ShapeDtypeStruct((T, H), jnp.float32),
        scratch_types=[
            pltpu.VMEM((HALF,), jnp.int32),
            pltpu.VMEM((HALF,), jnp.int32),
            pltpu.VMEM((HALF,), jnp.int32),
            pltpu.VMEM((HALF,), jnp.int32),
            pltpu.VMEM((HALF, H), jnp.float32),
            pltpu.VMEM((HALF, H), jnp.float32),
            pltpu.VMEM((2 * HALF, LANES), jnp.float32),
            pltpu.VMEM((2 * HALF, LANES), jnp.float32),
            pltpu.VMEM((HALF, H), jnp.float32),
            pltpu.SemaphoreType.DMA,
            pltpu.SemaphoreType.DMA,
        ],
    )


def kernel(hidden_states, W_gate, Wg, Wu, W2):
    pe2, po2, se, so, g2, nv8, b2, nx2 = _routing_call(hidden_states, W_gate)
    pe = pe2.reshape(T)
    po = po2.reshape(T)
    g = g2.reshape(NTILES)
    nv = nv8.reshape(8)[:1]
    b = b2.reshape(NTILES)
    nx = nx2.reshape(NTILES)
    x_sorted = _dispatch_kernel()(hidden_states, pe, po)
    out_sorted = _ffn_call(g, nv, b, nx, x_sorted, Wg, Wu, W2)
    return _combine_kernel()(out_sorted, pe, po, se, so)


# speedup vs baseline: 1.0058x; 1.0058x over previous
import functools

import jax
import jax.numpy as jnp
from jax import lax
from jax.experimental import pallas as pl
from jax.experimental.pallas import tpu as pltpu
from jax.experimental.pallas import tpu_sc as plsc

T = 2048
H = 1024
DFF = 2048
E = 8
TILE = 128
P = T * 2 + E * TILE
NTILES = P // TILE
NC, NS, LANES = 2, 16, 16
NW = NC * NS
TPW = T // NW
HALF = TPW // 2

_NEG_INF = float("-inf")


def _routing_body(hid_ref, wgate_ref, pe_ref, po_ref, se_ref, so_ref,
                  g_ref, nv_ref, b_ref, nx_ref):
    hid = hid_ref[...]
    logits = lax.dot_general(hid, wgate_ref[...], (((1,), (1,)), ((), ())),
                             preferred_element_type=jnp.float32)
    iota_e = lax.broadcasted_iota(jnp.int32, (T, E), 1)
    m1 = jnp.max(logits, axis=1, keepdims=True)
    am1 = jnp.min(jnp.where(logits == m1, iota_e, E), axis=1, keepdims=True)
    masked = jnp.where(iota_e == am1, _NEG_INF, logits)
    m2 = jnp.max(masked, axis=1, keepdims=True)
    am2 = jnp.min(jnp.where(masked == m2, iota_e, E), axis=1, keepdims=True)
    s1 = 1.0 / (1.0 + jnp.exp(m2 - m1))
    s2 = 1.0 - s1

    one_e = (iota_e == am1).astype(jnp.float32)
    one_o = (iota_e == am2).astype(jnp.float32)
    segment = one_e + one_o
    incl = segment
    shift = 1
    while shift < T:
        top = jnp.zeros((shift, E), jnp.float32)
        incl = incl + jnp.concatenate([top, incl[: T - shift]], axis=0)
        shift *= 2
    run_excl = incl - segment

    counts = jnp.sum(segment, axis=0, keepdims=True)
    pc = ((counts.astype(jnp.int32) + (TILE - 1)) // TILE) * TILE
    re8 = lax.broadcasted_iota(jnp.int32, (E, E), 0)
    ce8 = lax.broadcasted_iota(jnp.int32, (E, E), 1)
    upper = (re8 < ce8).astype(jnp.float32)
    pstart = lax.dot_general(pc.astype(jnp.float32), upper,
                             (((1,), (0,)), ((), ())),
                             preferred_element_type=jnp.float32)
    total = jnp.sum(pc)

    slot = pstart + run_excl
    pe_ref[...] = jnp.sum(one_e * slot, axis=1, keepdims=True).astype(jnp.int32)
    po_ref[...] = jnp.sum(one_o * slot, axis=1, keepdims=True).astype(jnp.int32)
    se_ref[...] = jnp.broadcast_to(s1, (T, LANES))
    so_ref[...] = jnp.broadcast_to(s2, (T, LANES))

    tstart = lax.broadcasted_iota(jnp.int32, (NTILES, E), 0) * TILE
    teff = jnp.minimum(tstart, total - TILE)
    ipstart = pstart.astype(jnp.int32)
    g = jnp.sum((teff >= ipstart).astype(jnp.int32), axis=1, keepdims=True) - 1
    g_ref[...] = g
    nv_ref[...] = jnp.full((8, 1), total // TILE, jnp.int32)

    iota_ne = lax.broadcasted_iota(jnp.int32, (NTILES, E), 1)
    used = jnp.broadcast_to((pc > 0).astype(jnp.int32), (NTILES, E))
    b_ref[...] = jnp.sum(used * (iota_ne < g).astype(jnp.int32),
                         axis=1, keepdims=True)
    cand = jnp.where((used > 0) & (iota_ne > g), iota_ne, E + 99)
    nxt = jnp.min(cand, axis=1, keepdims=True)
    nx_ref[...] = jnp.where(nxt == E + 99, -1, nxt)


def _routing_call(hidden_states, w_gate, interpret=False):
    return pl.pallas_call(
        _routing_body,
        out_shape=(
            jax.ShapeDtypeStruct((T, 1), jnp.int32),
            jax.ShapeDtypeStruct((T, 1), jnp.int32),
            jax.ShapeDtypeStruct((T, LANES), jnp.float32),
            jax.ShapeDtypeStruct((T, LANES), jnp.float32),
            jax.ShapeDtypeStruct((NTILES, 1), jnp.int32),
            jax.ShapeDtypeStruct((8, 1), jnp.int32),
            jax.ShapeDtypeStruct((NTILES, 1), jnp.int32),
            jax.ShapeDtypeStruct((NTILES, 1), jnp.int32),
        ),
        interpret=interpret,
    )(hidden_states, w_gate)


def _ffn_body(g_sref, nv_sref, b_sref, nx_sref,
              x_ref, wg_hbm, wu_hbm, w2_hbm, out_ref,
              wgb, wub, w2b, sg0, su0, s20, sg1, su1, s21):
    t = pl.program_id(0)
    nv = nv_sref[0]
    e = g_sref[t]
    ordb = b_sref[t]
    is_valid = t < nv
    prevb = b_sref[jnp.maximum(t - 1, 0)]
    at_boundary = jnp.logical_and(is_valid,
                                  jnp.logical_or(t == 0, ordb != prevb))
    sems = ((sg0, su0, s20), (sg1, su1, s21))

    def start_fetch(eid, k):
        pltpu.make_async_copy(wg_hbm.at[eid], wgb.at[k], sems[k][0]).start()
        pltpu.make_async_copy(wu_hbm.at[eid], wub.at[k], sems[k][1]).start()
        pltpu.make_async_copy(w2_hbm.at[eid], w2b.at[k], sems[k][2]).start()

    @pl.when(jnp.logical_and(t == 0, is_valid))
    def _():
        start_fetch(e, 0)

    for k in (0, 1):
        @pl.when(jnp.logical_and(at_boundary, lax.rem(ordb, 2) == k))
        def _(k=k):
            pltpu.make_async_copy(wg_hbm.at[e], wgb.at[k], sems[k][0]).wait()
            pltpu.make_async_copy(wu_hbm.at[e], wub.at[k], sems[k][1]).wait()
            pltpu.make_async_copy(w2_hbm.at[e], w2b.at[k], sems[k][2]).wait()
            ne = nx_sref[t]

            @pl.when(ne >= 0)
            def _():
                start_fetch(ne, 1 - k)

    for k in (0, 1):
        @pl.when(jnp.logical_and(is_valid, lax.rem(ordb, 2) == k))
        def _(k=k):
            x = x_ref[...]
            a = jnp.dot(x, wgb[k], preferred_element_type=jnp.float32)
            bb = jnp.dot(x, wub[k], preferred_element_type=jnp.float32)
            h = a * (1.0 / (1.0 + jnp.exp(-a))) * bb
            out_ref[...] = jnp.dot(h, w2b[k], preferred_element_type=jnp.float32)


def _ffn_call(g, nv, b, nx, x_sorted, wg, wu, w2, interpret=False):
    grid_spec = pltpu.PrefetchScalarGridSpec(
        num_scalar_prefetch=4,
        grid=(NTILES,),
        in_specs=[
            pl.BlockSpec((TILE, H),
                         lambda t, g, nv, b, nx: (jnp.minimum(t, nv[0] - 1), 0)),
            pl.BlockSpec(memory_space=pl.ANY),
            pl.BlockSpec(memory_space=pl.ANY),
            pl.BlockSpec(memory_space=pl.ANY),
        ],
        out_specs=pl.BlockSpec((TILE, H),
                               lambda t, g, nv, b, nx: (jnp.minimum(t, nv[0] - 1), 0)),
        scratch_shapes=[
            pltpu.VMEM((2, H, DFF), jnp.float32),
            pltpu.VMEM((2, H, DFF), jnp.float32),
            pltpu.VMEM((2, DFF, H), jnp.float32),
            pltpu.SemaphoreType.DMA,
            pltpu.SemaphoreType.DMA,
            pltpu.SemaphoreType.DMA,
            pltpu.SemaphoreType.DMA,
            pltpu.SemaphoreType.DMA,
            pltpu.SemaphoreType.DMA,
        ],
    )
    return pl.pallas_call(
        _ffn_body,
        grid_spec=grid_spec,
        out_shape=jax.ShapeDtypeStruct((P, H), jnp.float32),
        name="moe_ffn",
        compiler_params=pltpu.CompilerParams(
            dimension_semantics=("arbitrary",)),
        interpret=interpret,
    )(g, nv, b, nx, x_sorted, wg, wu, w2)


def _dispatch_body(hid_hbm, pe_hbm, po_hbm, xs_hbm, idx0, idx1, rows, sem):
    wid = lax.axis_index("s") * NC + lax.axis_index("c")
    base = wid * TPW
    pltpu.sync_copy(pe_hbm.at[pl.ds(base, TPW)], idx0)
    pltpu.sync_copy(po_hbm.at[pl.ds(base, TPW)], idx1)
    pltpu.sync_copy(hid_hbm.at[pl.ds(base, TPW)], rows)
    cp0 = pltpu.async_copy(rows, xs_hbm.at[idx0], sem)
    cp1 = pltpu.async_copy(rows, xs_hbm.at[idx1], sem)
    cp0.wait()
    cp1.wait()


@functools.cache
def _dispatch_kernel():
    return pl.kernel(
        _dispatch_body,
        mesh=plsc.VectorSubcoreMesh(core_axis_name="c", subcore_axis_name="s"),
        out_type=jax.ShapeDtypeStruct((P, H), jnp.float32),
        scratch_types=[
            pltpu.VMEM((TPW,), jnp.int32),
            pltpu.VMEM((TPW,), jnp.int32),
            pltpu.VMEM((TPW, H), jnp.float32),
            pltpu.SemaphoreType.DMA,
        ],
    )


def _combine_body(os_hbm, pe_hbm, po_hbm, se_hbm, so_hbm, out_hbm,
                  idx0a, idx1a, idx0b, idx1b, rows0, rows1, sev, sov, acc,
                  sem0, sem1):
    wid = lax.axis_index("s") * NC + lax.axis_index("c")
    ha = wid * TPW
    hb = ha + HALF
    pltpu.sync_copy(pe_hbm.at[pl.ds(ha, HALF)], idx0a)
    pltpu.sync_copy(po_hbm.at[pl.ds(ha, HALF)], idx1a)
    a0 = pltpu.async_copy(os_hbm.at[idx0a], rows0, sem0)
    a1 = pltpu.async_copy(os_hbm.at[idx1a], rows1, sem1)
    pltpu.sync_copy(pe_hbm.at[pl.ds(hb, HALF)], idx0b)
    pltpu.sync_copy(po_hbm.at[pl.ds(hb, HALF)], idx1b)
    pltpu.sync_copy(se_hbm.at[pl.ds(ha, 2 * HALF)], sev)
    pltpu.sync_copy(so_hbm.at[pl.ds(ha, 2 * HALF)], sov)

    def mul_into(dst, src, sc_ref, off, accumulate):
        def tok_body(tk, carry):
            s = sc_ref[off + tk]
            for cch in range(H // LANES):
                sl = pl.ds(cch * LANES, LANES)
                if accumulate:
                    dst[tk, sl] = dst[tk, sl] + src[tk, sl] * s
                else:
                    dst[tk, sl] = src[tk, sl] * s
            return carry
        lax.fori_loop(0, HALF, tok_body, 0)

    a0.wait()
    mul_into(acc, rows0, sev, 0, False)
    b0 = pltpu.async_copy(os_hbm.at[idx0b], rows0, sem0)
    a1.wait()
    mul_into(acc, rows1, sov, 0, True)
    b1 = pltpu.async_copy(os_hbm.at[idx1b], rows1, sem1)
    pltpu.sync_copy(acc, out_hbm.at[pl.ds(ha, HALF)])
    b0.wait()
    mul_into(acc, rows0, sev, HALF, False)
    b1.wait()
    mul_into(acc, rows1, sov, HALF, True)
    pltpu.sync_copy(acc, out_hbm.at[pl.ds(hb, HALF)])


@functools.cache
def _combine_kernel():
    return pl.kernel(
        _combine_body,
        mesh=plsc.VectorSubcoreMesh(core_axis_name="c", subcore_axis_name="s"),
        out_type=jax.ShapeDtypeStruct((T, H), jnp.float32),
        scratch_types=[
            pltpu.VMEM((HALF,), jnp.int32),
            pltpu.VMEM((HALF,), jnp.int32),
            pltpu.VMEM((HALF,), jnp.int32),
            pltpu.VMEM((HALF,), jnp.int32),
            pltpu.VMEM((HALF, H), jnp.float32),
            pltpu.VMEM((HALF, H), jnp.float32),
            pltpu.VMEM((2 * HALF, LANES), jnp.float32),
            pltpu.VMEM((2 * HALF, LANES), jnp.float32),
            pltpu.VMEM((HALF, H), jnp.float32),
            pltpu.SemaphoreType.DMA,
            pltpu.SemaphoreType.DMA,
        ],
    )


def kernel(hidden_states, W_gate, Wg, Wu, W2):
    pe2, po2, se, so, g2, nv8, b2, nx2 = _routing_call(hidden_states, W_gate)
    pe = pe2.reshape(T)
    po = po2.reshape(T)
    g = g2.reshape(NTILES)
    nv = nv8.reshape(8)[:1]
    b = b2.reshape(NTILES)
    nx = nx2.reshape(NTILES)
    x_sorted = _dispatch_kernel()(hidden_states, pe, po)
    out_sorted = _ffn_call(g, nv, b, nx, x_sorted, Wg, Wu, W2)
    return _combine_kernel()(out_sorted, pe, po, se, so)
